# baseline (speedup 1.0000x reference)
import jax
import jax.numpy as jnp
from jax.experimental import pallas as pl
from jax.experimental.pallas import tpu as pltpu

LANES = 8192
ROWS = 24
NSUB = 8
NPRI = 2


def _pos_body(xe_ref, ye_ref, pos_ref):
    pos_ref[...] = xe_ref[...][:, None, :] + ye_ref[...][None, :, :]


def _add_body(x_ref, pos_ref, o_ref, ibuf, obuf, isem, osem):
    i = pl.program_id(0)
    nsteps = pl.num_programs(0)
    ph = jax.lax.rem(i, 2)

    def in_copy(step, phase, q):
        return pltpu.make_async_copy(
            x_ref.at[pl.ds((step * NSUB + q) * ROWS, ROWS)],
            ibuf.at[phase, q],
            isem.at[phase, q],
        )

    def out_copy(step, phase, q):
        return pltpu.make_async_copy(
            obuf.at[phase, q],
            o_ref.at[pl.ds((step * NSUB + q) * ROWS, ROWS)],
            osem.at[phase, q],
        )

    @pl.when(i == 0)
    def _prologue():
        for q in range(NSUB):
            in_copy(0, 0, q).start(priority=q % NPRI)
        for q in range(NSUB):
            in_copy(1, 1, q).start(priority=q % NPRI)

    for q in range(NSUB):
        in_copy(i, ph, q).wait()

    @pl.when(i >= 2)
    def _wait_prev_out():
        for q in range(NSUB):
            out_copy(i - 2, ph, q).wait()

    obuf[ph] = ibuf[ph] + pos_ref[...][None]

    for q in range(NSUB):
        out_copy(i, ph, q).start(priority=q % NPRI)

    @pl.when(i + 2 < nsteps)
    def _prefetch():
        for q in range(NSUB):
            in_copy(i + 2, ph, q).start(priority=q % NPRI)

    @pl.when(i == nsteps - 1)
    def _drain():
        for phase in range(2):
            for q in range(NSUB):
                pltpu.make_async_copy(
                    obuf.at[phase, q],
                    o_ref.at[pl.ds(q * ROWS, ROWS)],
                    osem.at[phase, q],
                ).wait()


def kernel(x, xemb, yemb):
    B, H, W, D = x.shape

    posemb = pl.pallas_call(
        _pos_body,
        out_shape=jax.ShapeDtypeStruct((H, W, D), x.dtype),
    )(xemb, yemb)

    pos2 = posemb.reshape(ROWS, LANES)
    x2 = x.reshape(B * ROWS, LANES)
    nsteps = (B * ROWS) // (NSUB * ROWS)

    out = pl.pallas_call(
        _add_body,
        grid=(nsteps,),
        in_specs=[
            pl.BlockSpec(memory_space=pltpu.MemorySpace.HBM),
            pl.BlockSpec((ROWS, LANES), lambda i: (0, 0)),
        ],
        out_specs=pl.BlockSpec(memory_space=pltpu.MemorySpace.HBM),
        out_shape=jax.ShapeDtypeStruct((B * ROWS, LANES), x.dtype),
        scratch_shapes=[
            pltpu.VMEM((2, NSUB, ROWS, LANES), x.dtype),
            pltpu.VMEM((2, NSUB, ROWS, LANES), x.dtype),
            pltpu.SemaphoreType.DMA((2, NSUB)),
            pltpu.SemaphoreType.DMA((2, NSUB)),
        ],
        compiler_params=pltpu.CompilerParams(
            dimension_semantics=("arbitrary",),
        ),
    )(x2, pos2)
    return out.reshape(B, H, W, D)
